# baseline (speedup 1.0000x reference)
import jax
import jax.numpy as jnp
from jax import lax
from jax.experimental import pallas as pl
from jax.experimental.pallas import tpu as pltpu
from jax.experimental.pallas import tpu_sc as plsc

M = 6144000
D = 8
B = 2000000

NC = 2
NS = 16
L = 16

RPB = 131072
NBINS = 47
BIN_SHIFT = 17

SPAN = 62528
HALF = SPAN // 2
NVEC = HALF // L
T31 = B - 31 * SPAN
H31 = T31 // 2
NV31 = H31 // L
B_PAD = SPAN * 32
NSEG = 64

PIECE = 512
PIECE_SHIFT = 9
LCAP = B_PAD + NSEG * NBINS * (PIECE - 1) + PIECE
LCAP = ((LCAP + PIECE - 1) // PIECE) * PIECE

LOC_CAP = HALF + NBINS * 63 + PIECE

_mesh = plsc.VectorSubcoreMesh(
    core_axis_name="c", subcore_axis_name="s", num_cores=NC, num_subcores=NS
)
_cp = pltpu.CompilerParams(
    needs_layout_passes=False, use_tc_tiling_on_sc=False
)

_ZV = lambda: jnp.zeros((L,), jnp.int32)
_ONES = lambda: jnp.ones((L,), jnp.int32)


def _lane():
  return lax.broadcasted_iota(jnp.int32, (L,), 0)


def _al(x, m=8):
  return pl.multiple_of(x, m)


def _sel(ref, e):
  return jnp.max(plsc.load_gather(ref, [_ZV() + e]))


def _sel_s(ref, e):
  v = ref[pl.ds((e // L) * L, L)]
  return jnp.max(jnp.where(_lane() == (e % L), v, 0))


def _k1_body(idx_hbm, counts_hbm, chunk, hist, sem):
  c = lax.axis_index("c")
  s = lax.axis_index("s")
  w = s * NC + c
  for j in range(6):
    hist[pl.ds(j * L, L)] = _ZV()
  w31 = w == 31
  for h in range(2):
    @pl.when(w31)
    def _():
      pltpu.sync_copy(idx_hbm.at[pl.ds(_al(31 * SPAN + h * H31), H31)],
                      chunk.at[pl.ds(0, H31)])

    @pl.when(jnp.logical_not(w31))
    def _():
      pltpu.sync_copy(idx_hbm.at[pl.ds(_al(w * SPAN + h * HALF), HALF)], chunk)

    base = jnp.int32(h * (NBINS + 1))

    def body(i, carry):
      v = chunk[pl.ds(i * L, L)]
      binv = lax.shift_right_logical(v, BIN_SHIFT) + base
      plsc.addupdate_scatter(hist, [binv], _ONES())
      return carry

    lax.fori_loop(0, jnp.where(w31, NV31, NVEC), body, jnp.int32(0))
  pltpu.sync_copy(hist, counts_hbm.at[pl.ds(_al(w * 2 * (NBINS + 1)), 96)])


_k1 = pl.kernel(
    _k1_body,
    out_type=jax.ShapeDtypeStruct((NSEG * (NBINS + 1),), jnp.int32),
    mesh=_mesh,
    scratch_types=[
        pltpu.VMEM((HALF,), jnp.int32),
        pltpu.VMEM((96,), jnp.int32),
        pltpu.SemaphoreType.DMA,
    ],
    compiler_params=_cp,
)


def _padpv(v):
  return lax.shift_left(
      lax.shift_right_logical(v + (PIECE - 1), PIECE_SHIFT), PIECE_SHIFT)


def _pad64v(v):
  return lax.shift_left(lax.shift_right_logical(v + 63, 6), 6)


def _excl_prefix48(vecs):
  out = []
  carry = jnp.int32(0)
  for v in vecs:
    incl = plsc.cumsum(v)
    out.append(incl - v + carry)
    carry = carry + jnp.sum(v)
  return out


def _k2_body(idx_hbm, counts_hbm, lpid_hbm, lidx_hbm, segoff_hbm,
             chunk, lpid, lidx, cbuf, hist, cursors, localoff, sobuf, sem):
  c = lax.axis_index("c")
  s = lax.axis_index("s")
  w = s * NC + c
  lane = _lane()

  pltpu.sync_copy(counts_hbm, cbuf)
  acc = [_ZV(), _ZV(), _ZV()]
  mine = [[None] * 3, [None] * 3]
  for sg in range(NSEG):
    is_mine0 = (_ZV() + (2 * w)) == sg
    is_mine1 = (_ZV() + (2 * w + 1)) == sg
    for j in range(3):
      r = cbuf[pl.ds(sg * 48 + j * L, L)]
      p = _padpv(r)
      m0 = jnp.where(is_mine0, acc[j], _ZV())
      m1 = jnp.where(is_mine1, acc[j], _ZV())
      mine[0][j] = m0 if mine[0][j] is None else mine[0][j] + m0
      mine[1][j] = m1 if mine[1][j] is None else mine[1][j] + m1
      acc[j] = acc[j] + p
  base3 = _excl_prefix48(acc)
  for h in range(2):
    for j in range(3):
      sobuf[pl.ds(h * 48 + j * L, L)] = base3[j] + mine[h][j]
  pltpu.sync_copy(sobuf, segoff_hbm.at[pl.ds(_al(w * 96), 96)])

  drain = pltpu.make_async_copy(
      lpid_hbm.at[pl.ds(0, PIECE)], chunk.at[pl.ds(0, PIECE)], sem
  )

  n_out = jnp.int32(0)
  w31 = w == 31
  for h in range(2):
    @pl.when(w31)
    def _():
      pltpu.sync_copy(idx_hbm.at[pl.ds(_al(31 * SPAN + h * H31), H31)],
                      chunk.at[pl.ds(0, H31)])

    @pl.when(jnp.logical_not(w31))
    def _():
      pltpu.sync_copy(idx_hbm.at[pl.ds(_al(w * SPAN + h * HALF), HALF)], chunk)

    pidbase = jnp.where(w31, 31 * SPAN + h * H31, w * SPAN + h * HALF)

    for j in range(3):
      hist[pl.ds(j * L, L)] = _ZV()

    def cbody(i, carry):
      v = chunk[pl.ds(i * L, L)]
      binv = lax.shift_right_logical(v, BIN_SHIFT)
      plsc.addupdate_scatter(hist, [binv], _ONES())
      return carry

    lax.fori_loop(0, jnp.where(w31, NV31, NVEC), cbody, jnp.int32(0))

    hv = [_pad64v(hist[pl.ds(j * L, L)]) for j in range(3)]
    lo3 = _excl_prefix48(hv)
    for j in range(3):
      localoff[pl.ds(j * L, L)] = lo3[j]
      cursors[pl.ds(j * L, L)] = lo3[j]

    def sbody(i, carry):
      v = chunk[pl.ds(i * L, L)]
      binv = lax.shift_right_logical(v, BIN_SHIFT)
      rank, _ = plsc.scan_count(binv)
      cur = plsc.load_gather(cursors, [binv])
      pos = cur + rank - 1
      pidv = lane + (pidbase + i * L)
      plsc.store_scatter(lpid, [pos], pidv)
      plsc.store_scatter(lidx, [pos], v)
      plsc.addupdate_scatter(cursors, [binv], _ONES())
      return carry

    lax.fori_loop(0, jnp.where(w31, NV31, NVEC), sbody, jnp.int32(0))

    for b in range(NBINS):
      cnt = _sel_s(hist, b)
      lo = _sel_s(localoff, b)
      gs = _sel_s(sobuf, h * 48 + b)
      npc = lax.shift_right_logical(cnt + (PIECE - 1), PIECE_SHIFT)

      def fbody(i, carry):
        src_off = _al(lo + i * PIECE)
        dst_off = _al(gs + i * PIECE)
        pltpu.async_copy(
            lpid.at[pl.ds(src_off, PIECE)],
            lpid_hbm.at[pl.ds(dst_off, PIECE)], sem)
        pltpu.async_copy(
            lidx.at[pl.ds(src_off, PIECE)],
            lidx_hbm.at[pl.ds(dst_off, PIECE)], sem)
        o = carry + 2

        @pl.when(o >= 8)
        def _():
          drain.wait()
          drain.wait()

        return jnp.where(o >= 8, o - 2, o)

      n_out = lax.fori_loop(0, npc, fbody, n_out)

    def dbody(i, carry):
      drain.wait()
      return carry

    lax.fori_loop(0, n_out, dbody, jnp.int32(0))
    n_out = jnp.int32(0)


_k2 = pl.kernel(
    _k2_body,
    out_type=(
        jax.ShapeDtypeStruct((LCAP,), jnp.int32),
        jax.ShapeDtypeStruct((LCAP,), jnp.int32),
        jax.ShapeDtypeStruct((NSEG * 96 // 2,), jnp.int32),
    ),
    mesh=_mesh,
    scratch_types=[
        pltpu.VMEM((HALF,), jnp.int32),
        pltpu.VMEM((LOC_CAP,), jnp.int32),
        pltpu.VMEM((LOC_CAP,), jnp.int32),
        pltpu.VMEM((NSEG * 48,), jnp.int32),
        pltpu.VMEM((48,), jnp.int32),
        pltpu.VMEM((48,), jnp.int32),
        pltpu.VMEM((48,), jnp.int32),
        pltpu.VMEM((96,), jnp.int32),
        pltpu.SemaphoreType.DMA,
    ],
    compiler_params=_cp,
)


def _k3_seg(b, sg, cbuf, sobuf, lpid_hbm, lidx_hbm, val_hbm,
            pbuf, ibuf, vrows, acc, sem, lane):
  e = sg * 48 + b
  cnt = _sel(cbuf, e)
  gs = _sel(sobuf, e)
  nch = lax.shift_right_logical(cnt + (PIECE - 1), PIECE_SHIFT)
  rowbase = b * RPB

  def chunk_body(i, carry):
    off = _al(gs + i * PIECE)
    cp1 = pltpu.async_copy(lpid_hbm.at[pl.ds(off, PIECE)], pbuf, sem)
    cp2 = pltpu.async_copy(lidx_hbm.at[pl.ds(off, PIECE)], ibuf, sem)
    cp1.wait()
    cp2.wait()
    valid = cnt - i * PIECE
    for v in range(PIECE // L):
      lg = lane + v * L
      m = lg < valid
      pv = jnp.where(m, pbuf[pl.ds(v * L, L)], jnp.int32(0))
      pbuf[pl.ds(v * L, L)] = jnp.clip(pv, 0, B - 1)
      iv = ibuf[pl.ds(v * L, L)]
      rv = jnp.where(m, iv - rowbase, jnp.int32(RPB))
      ibuf[pl.ds(v * L, L)] = jnp.clip(rv, 0, RPB)
    pltpu.async_copy(val_hbm.at[pbuf], vrows, sem).wait()
    pltpu.sync_copy(vrows, acc.at[ibuf], add=True)
    return carry

  lax.fori_loop(0, nch, chunk_body, jnp.int32(0))


def _k3_bin(b, nrows, mem_hbm, out_hbm, lpid_hbm, lidx_hbm, val_hbm,
            cbuf, sobuf, pbuf, ibuf, vrows, acc, sem, s, lane):
  chr_ = nrows // NS
  row0 = _al(b * RPB + s * chr_)
  pltpu.sync_copy(mem_hbm.at[pl.ds(row0, chr_)], acc.at[pl.ds(_al(s * chr_), chr_)])
  plsc.subcore_barrier()
  for t in range(4):
    _k3_seg(b, s * 4 + t, cbuf, sobuf, lpid_hbm, lidx_hbm, val_hbm,
            pbuf, ibuf, vrows, acc, sem, lane)
  plsc.subcore_barrier()
  pltpu.sync_copy(acc.at[pl.ds(_al(s * chr_), chr_)], out_hbm.at[pl.ds(row0, chr_)])
  plsc.subcore_barrier()


def _k3_body(mem_hbm, val_hbm, counts_hbm, segoff_hbm, lpid_hbm, lidx_hbm,
             out_hbm, cbuf, sobuf, pbuf, ibuf, vrows, acc, sem):
  c = lax.axis_index("c")
  s = lax.axis_index("s")
  lane = _lane()
  pltpu.sync_copy(counts_hbm, cbuf)
  pltpu.sync_copy(segoff_hbm, sobuf)

  args = (mem_hbm, out_hbm, lpid_hbm, lidx_hbm, val_hbm,
          cbuf, sobuf, pbuf, ibuf, vrows, acc, sem, s, lane)

  @pl.when(c == 0)
  def _():
    def body(i, carry):
      _k3_bin(i, RPB, *args)
      return carry

    lax.fori_loop(0, 24, body, jnp.int32(0))

  @pl.when(c == 1)
  def _():
    def body(i, carry):
      _k3_bin(24 + i, RPB, *args)
      return carry

    lax.fori_loop(0, 22, body, jnp.int32(0))
    _k3_bin(jnp.int32(46), M - 46 * RPB, *args)


_k3 = pl.kernel(
    _k3_body,
    out_type=jax.ShapeDtypeStruct((M, D), jnp.float32),
    mesh=_mesh,
    scratch_types=[
        pltpu.VMEM((NSEG * 48,), jnp.int32),
        pltpu.VMEM((NSEG * 48,), jnp.int32),
        pltpu.VMEM((PIECE,), jnp.int32),
        pltpu.VMEM((PIECE,), jnp.int32),
        pltpu.VMEM((PIECE, D), jnp.float32),
        pltpu.VMEM_SHARED((RPB + 8, D), jnp.float32),
        pltpu.SemaphoreType.DMA,
    ],
    compiler_params=_cp,
)


def kernel(mem, idx, val):
  idx32 = idx.astype(jnp.int32)
  counts = _k1(idx32)
  lpid, lidx, segoff = _k2(idx32, counts)
  return _k3(mem, val, counts, segoff, lpid, lidx)


# speedup vs baseline: 1.7071x; 1.7071x over previous
import jax
import jax.numpy as jnp
from jax import lax
from jax.experimental import pallas as pl
from jax.experimental.pallas import tpu as pltpu
from jax.experimental.pallas import tpu_sc as plsc

M = 6144000
D = 8
B = 2000000

NC = 2
NS = 16
L = 16

RPB = 131072
NBINS = 47
BIN_SHIFT = 17

SPAN = 62528
HALF = SPAN // 2
NVEC = HALF // L
T31 = B - 31 * SPAN
H31 = T31 // 2
NV31 = H31 // L
B_PAD = SPAN * 32
NSEG = 64

PIECE = 256
PIECE_SHIFT = 8
LCAP = B_PAD + NSEG * NBINS * (PIECE - 1) + PIECE
LCAP = ((LCAP + PIECE - 1) // PIECE) * PIECE

LOC_CAP = HALF + NBINS * 63 + PIECE

_mesh = plsc.VectorSubcoreMesh(
    core_axis_name="c", subcore_axis_name="s", num_cores=NC, num_subcores=NS
)
_cp = pltpu.CompilerParams(
    needs_layout_passes=False, use_tc_tiling_on_sc=False
)

_ZV = lambda: jnp.zeros((L,), jnp.int32)
_ONES = lambda: jnp.ones((L,), jnp.int32)


def _lane():
  return lax.broadcasted_iota(jnp.int32, (L,), 0)


def _al(x, m=8):
  return pl.multiple_of(x, m)


def _sel(ref, e):
  return jnp.max(plsc.load_gather(ref, [_ZV() + e]))


def _sel_s(ref, e):
  v = ref[pl.ds((e // L) * L, L)]
  return jnp.max(jnp.where(_lane() == (e % L), v, 0))


def _k1_body(idx_hbm, counts_hbm, chunk, hist, sem):
  c = lax.axis_index("c")
  s = lax.axis_index("s")
  w = s * NC + c
  for j in range(6):
    hist[pl.ds(j * L, L)] = _ZV()
  w31 = w == 31
  for h in range(2):
    @pl.when(w31)
    def _():
      pltpu.sync_copy(idx_hbm.at[pl.ds(_al(31 * SPAN + h * H31), H31)],
                      chunk.at[pl.ds(0, H31)])

    @pl.when(jnp.logical_not(w31))
    def _():
      pltpu.sync_copy(idx_hbm.at[pl.ds(_al(w * SPAN + h * HALF), HALF)], chunk)

    base = jnp.int32(h * (NBINS + 1))

    def body(i, carry):
      v = chunk[pl.ds(i * L, L)]
      binv = lax.shift_right_logical(v, BIN_SHIFT) + base
      plsc.addupdate_scatter(hist, [binv], _ONES())
      return carry

    lax.fori_loop(0, jnp.where(w31, NV31, NVEC), body, jnp.int32(0))
  pltpu.sync_copy(hist, counts_hbm.at[pl.ds(_al(w * 2 * (NBINS + 1)), 96)])


_k1 = pl.kernel(
    _k1_body,
    out_type=jax.ShapeDtypeStruct((NSEG * (NBINS + 1),), jnp.int32),
    mesh=_mesh,
    scratch_types=[
        pltpu.VMEM((HALF,), jnp.int32),
        pltpu.VMEM((96,), jnp.int32),
        pltpu.SemaphoreType.DMA,
    ],
    compiler_params=_cp,
)


def _padpv(v):
  return lax.shift_left(
      lax.shift_right_logical(v + (PIECE - 1), PIECE_SHIFT), PIECE_SHIFT)


def _pad64v(v):
  return lax.shift_left(lax.shift_right_logical(v + 63, 6), 6)


def _excl_prefix48(vecs):
  out = []
  carry = jnp.int32(0)
  for v in vecs:
    incl = plsc.cumsum(v)
    out.append(incl - v + carry)
    carry = carry + jnp.sum(v)
  return out


def _k2_body(idx_hbm, counts_hbm, lpid_hbm, lidx_hbm, segoff_hbm,
             chunk, lpid, lidx, cbuf, hist, cursors, localoff, sobuf, sem):
  c = lax.axis_index("c")
  s = lax.axis_index("s")
  w = s * NC + c
  lane = _lane()

  pltpu.sync_copy(counts_hbm, cbuf)
  acc = [_ZV(), _ZV(), _ZV()]
  mine = [[None] * 3, [None] * 3]
  for sg in range(NSEG):
    is_mine0 = (_ZV() + (2 * w)) == sg
    is_mine1 = (_ZV() + (2 * w + 1)) == sg
    for j in range(3):
      r = cbuf[pl.ds(sg * 48 + j * L, L)]
      p = _padpv(r)
      m0 = jnp.where(is_mine0, acc[j], _ZV())
      m1 = jnp.where(is_mine1, acc[j], _ZV())
      mine[0][j] = m0 if mine[0][j] is None else mine[0][j] + m0
      mine[1][j] = m1 if mine[1][j] is None else mine[1][j] + m1
      acc[j] = acc[j] + p
  base3 = _excl_prefix48(acc)
  for h in range(2):
    for j in range(3):
      sobuf[pl.ds(h * 48 + j * L, L)] = base3[j] + mine[h][j]
  pltpu.sync_copy(sobuf, segoff_hbm.at[pl.ds(_al(w * 96), 96)])

  drain = pltpu.make_async_copy(
      lpid_hbm.at[pl.ds(0, PIECE)], chunk.at[pl.ds(0, PIECE)], sem
  )

  n_out = jnp.int32(0)
  w31 = w == 31
  for h in range(2):
    @pl.when(w31)
    def _():
      pltpu.sync_copy(idx_hbm.at[pl.ds(_al(31 * SPAN + h * H31), H31)],
                      chunk.at[pl.ds(0, H31)])

    @pl.when(jnp.logical_not(w31))
    def _():
      pltpu.sync_copy(idx_hbm.at[pl.ds(_al(w * SPAN + h * HALF), HALF)], chunk)

    pidbase = jnp.where(w31, 31 * SPAN + h * H31, w * SPAN + h * HALF)

    for j in range(3):
      hist[pl.ds(j * L, L)] = _ZV()

    def cbody(i, carry):
      v = chunk[pl.ds(i * L, L)]
      binv = lax.shift_right_logical(v, BIN_SHIFT)
      plsc.addupdate_scatter(hist, [binv], _ONES())
      return carry

    lax.fori_loop(0, jnp.where(w31, NV31, NVEC), cbody, jnp.int32(0))

    hv = [_pad64v(hist[pl.ds(j * L, L)]) for j in range(3)]
    lo3 = _excl_prefix48(hv)
    for j in range(3):
      localoff[pl.ds(j * L, L)] = lo3[j]
      cursors[pl.ds(j * L, L)] = lo3[j]

    def sbody(i, carry):
      v = chunk[pl.ds(i * L, L)]
      binv = lax.shift_right_logical(v, BIN_SHIFT)
      rank, _ = plsc.scan_count(binv)
      cur = plsc.load_gather(cursors, [binv])
      pos = cur + rank - 1
      pidv = lane + (pidbase + i * L)
      plsc.store_scatter(lpid, [pos], pidv)
      plsc.store_scatter(lidx, [pos], v)
      plsc.addupdate_scatter(cursors, [binv], _ONES())
      return carry

    lax.fori_loop(0, jnp.where(w31, NV31, NVEC), sbody, jnp.int32(0))

    for b in range(NBINS):
      cnt = _sel_s(hist, b)
      lo = _sel_s(localoff, b)
      gs = _sel_s(sobuf, h * 48 + b)
      npc = lax.shift_right_logical(cnt + (PIECE - 1), PIECE_SHIFT)

      def fbody(i, carry):
        src_off = _al(lo + i * PIECE)
        dst_off = _al(gs + i * PIECE)
        pltpu.async_copy(
            lpid.at[pl.ds(src_off, PIECE)],
            lpid_hbm.at[pl.ds(dst_off, PIECE)], sem)
        pltpu.async_copy(
            lidx.at[pl.ds(src_off, PIECE)],
            lidx_hbm.at[pl.ds(dst_off, PIECE)], sem)
        o = carry + 2

        @pl.when(o >= 8)
        def _():
          drain.wait()
          drain.wait()

        return jnp.where(o >= 8, o - 2, o)

      n_out = lax.fori_loop(0, npc, fbody, n_out)

    def dbody(i, carry):
      drain.wait()
      return carry

    lax.fori_loop(0, n_out, dbody, jnp.int32(0))
    n_out = jnp.int32(0)


_k2 = pl.kernel(
    _k2_body,
    out_type=(
        jax.ShapeDtypeStruct((LCAP,), jnp.int32),
        jax.ShapeDtypeStruct((LCAP,), jnp.int32),
        jax.ShapeDtypeStruct((NSEG * 96 // 2,), jnp.int32),
    ),
    mesh=_mesh,
    scratch_types=[
        pltpu.VMEM((HALF,), jnp.int32),
        pltpu.VMEM((LOC_CAP,), jnp.int32),
        pltpu.VMEM((LOC_CAP,), jnp.int32),
        pltpu.VMEM((NSEG * 48,), jnp.int32),
        pltpu.VMEM((48,), jnp.int32),
        pltpu.VMEM((48,), jnp.int32),
        pltpu.VMEM((48,), jnp.int32),
        pltpu.VMEM((96,), jnp.int32),
        pltpu.SemaphoreType.DMA,
    ],
    compiler_params=_cp,
)


def _k3_seg(b, sg, cbuf, sobuf, lpid_hbm, lidx_hbm, val_hbm,
            pbuf, ibuf, vrows, acc, sem, lane):
  e = sg * 48 + b
  cnt = _sel(cbuf, e)
  gs = _sel(sobuf, e)
  nch = lax.shift_right_logical(cnt + (PIECE - 1), PIECE_SHIFT)
  rowbase = b * RPB

  def chunk_body(i, carry):
    off = _al(gs + i * PIECE)
    cp1 = pltpu.async_copy(lpid_hbm.at[pl.ds(off, PIECE)], pbuf, sem)
    cp2 = pltpu.async_copy(lidx_hbm.at[pl.ds(off, PIECE)], ibuf, sem)
    cp1.wait()
    cp2.wait()
    valid = cnt - i * PIECE
    for v in range(PIECE // L):
      lg = lane + v * L
      m = lg < valid
      pv = jnp.where(m, pbuf[pl.ds(v * L, L)], lg * 61 + 17)
      pbuf[pl.ds(v * L, L)] = jnp.clip(pv, 0, B - 1)
      iv = ibuf[pl.ds(v * L, L)]
      rv = jnp.where(m, iv - rowbase, jnp.int32(RPB))
      ibuf[pl.ds(v * L, L)] = jnp.clip(rv, 0, RPB)
    pltpu.async_copy(val_hbm.at[pbuf], vrows, sem).wait()
    pltpu.sync_copy(vrows, acc.at[ibuf], add=True)
    return carry

  lax.fori_loop(0, nch, chunk_body, jnp.int32(0))


def _k3_bin(b, nrows, mem_hbm, out_hbm, lpid_hbm, lidx_hbm, val_hbm,
            cbuf, sobuf, pbuf, ibuf, vrows, acc, sem, s, lane):
  chr_ = nrows // NS
  row0 = _al(b * RPB + s * chr_)
  pltpu.sync_copy(mem_hbm.at[pl.ds(row0, chr_)], acc.at[pl.ds(_al(s * chr_), chr_)])
  plsc.subcore_barrier()
  for t in range(4):
    _k3_seg(b, s * 4 + t, cbuf, sobuf, lpid_hbm, lidx_hbm, val_hbm,
            pbuf, ibuf, vrows, acc, sem, lane)
  plsc.subcore_barrier()
  pltpu.sync_copy(acc.at[pl.ds(_al(s * chr_), chr_)], out_hbm.at[pl.ds(row0, chr_)])
  plsc.subcore_barrier()


def _k3_body(mem_hbm, val_hbm, counts_hbm, segoff_hbm, lpid_hbm, lidx_hbm,
             out_hbm, cbuf, sobuf, pbuf, ibuf, vrows, acc, sem):
  c = lax.axis_index("c")
  s = lax.axis_index("s")
  lane = _lane()
  pltpu.sync_copy(counts_hbm, cbuf)
  pltpu.sync_copy(segoff_hbm, sobuf)

  args = (mem_hbm, out_hbm, lpid_hbm, lidx_hbm, val_hbm,
          cbuf, sobuf, pbuf, ibuf, vrows, acc, sem, s, lane)

  @pl.when(c == 0)
  def _():
    def body(i, carry):
      _k3_bin(i, RPB, *args)
      return carry

    lax.fori_loop(0, 24, body, jnp.int32(0))

  @pl.when(c == 1)
  def _():
    def body(i, carry):
      _k3_bin(24 + i, RPB, *args)
      return carry

    lax.fori_loop(0, 22, body, jnp.int32(0))
    _k3_bin(jnp.int32(46), M - 46 * RPB, *args)


_k3 = pl.kernel(
    _k3_body,
    out_type=jax.ShapeDtypeStruct((M, D), jnp.float32),
    mesh=_mesh,
    scratch_types=[
        pltpu.VMEM((NSEG * 48,), jnp.int32),
        pltpu.VMEM((NSEG * 48,), jnp.int32),
        pltpu.VMEM((PIECE,), jnp.int32),
        pltpu.VMEM((PIECE,), jnp.int32),
        pltpu.VMEM((PIECE, D), jnp.float32),
        pltpu.VMEM_SHARED((RPB + 8, D), jnp.float32),
        pltpu.SemaphoreType.DMA,
    ],
    compiler_params=_cp,
)


def kernel(mem, idx, val):
  idx32 = idx.astype(jnp.int32)
  counts = _k1(idx32)
  lpid, lidx, segoff = _k2(idx32, counts)
  return _k3(mem, val, counts, segoff, lpid, lidx)
